# Initial kernel scaffold; baseline (speedup 1.0000x reference)
import jax
import jax.numpy as jnp
from jax.experimental import pallas as pl


def kernel(boxes, scores):
    raise NotImplementedError("write your pallas kernel here")


# speedup vs baseline: 10.0878x; 10.0878x over previous
import functools

import jax
import jax.numpy as jnp
from jax import lax
from jax.experimental import pallas as pl
from jax.experimental.pallas import tpu as pltpu
from jax.experimental.pallas import tpu_sc as plsc

N = 5000
MAX_OUT = 100
IOU_THRESH = 0.5
NTILES = 16
NPAD = 5120
PER_TILE = NPAD // NTILES
CHUNKS = PER_TILE // 16
NEG_INF = float("-inf")
BIG = 3.0e38


def _nms_body(bt_hbm, sp_hbm, out_hbm,
              cx1, cy1, cx2, cy2, areas, ms, stage, comb, outv, shared):
    tid = lax.axis_index("s")
    base = tid * PER_TILE

    pltpu.sync_copy(bt_hbm.at[0], cx1)
    pltpu.sync_copy(bt_hbm.at[1], cy1)
    pltpu.sync_copy(bt_hbm.at[2], cx2)
    pltpu.sync_copy(bt_hbm.at[3], cy2)
    pltpu.sync_copy(sp_hbm.at[pl.ds(base, PER_TILE)], ms)

    for j in range(CHUNKS):
        goff = base + j * 16
        a = (cx2[pl.ds(goff, 16)] - cx1[pl.ds(goff, 16)]) * (
            cy2[pl.ds(goff, 16)] - cy1[pl.ds(goff, 16)])
        areas[pl.ds(j * 16, 16)] = a

    li = lax.iota(jnp.int32, 16)
    lif = li.astype(jnp.float32)
    basef = base.astype(jnp.float32)

    def step(s, carry):
        ipf, b0, b1, b2, b3 = carry
        ai = (b2 - b0) * (b3 - b1)

        bestm = jnp.full((16,), NEG_INF, jnp.float32)
        besti = jnp.zeros((16,), jnp.float32)
        for j in range(CHUNKS):
            off = j * 16
            goff = base + off
            v = ms[pl.ds(off, 16)]
            c1 = cx1[pl.ds(goff, 16)]
            c2 = cy1[pl.ds(goff, 16)]
            c3 = cx2[pl.ds(goff, 16)]
            c4 = cy2[pl.ds(goff, 16)]
            ar = areas[pl.ds(off, 16)]
            xx1 = jnp.maximum(b0, c1)
            yy1 = jnp.maximum(b1, c2)
            xx2 = jnp.minimum(b2, c3)
            yy2 = jnp.minimum(b3, c4)
            inter = jnp.maximum(xx2 - xx1, 0.0) * jnp.maximum(yy2 - yy1, 0.0)
            iou = inter / (ai + ar - inter + jnp.float32(1e-8))
            idxv = lif + (basef + jnp.float32(off))
            keep = (iou <= IOU_THRESH) & (idxv != ipf)
            v2 = jnp.where(keep, v, NEG_INF)
            ms[pl.ds(off, 16)] = v2
            better = v2 > bestm
            bestm = jnp.where(better, v2, bestm)
            besti = jnp.where(better, idxv, besti)

        mloc = jnp.max(bestm)
        iloc = jnp.min(jnp.where(bestm == mloc, besti, BIG))

        stage[pl.ds(0, 16)] = jnp.where(li == tid, mloc, NEG_INF)
        stage[pl.ds(16, 16)] = jnp.where(li == tid, iloc, 0.0)
        p = lax.rem(s, 2)
        pltpu.sync_copy(stage, shared.at[pl.ds(p * 512 + tid * 32, 32)])
        plsc.subcore_barrier()
        pltpu.sync_copy(shared.at[pl.ds(p * 512, 512)], comb)

        mvec = jnp.full((16,), NEG_INF, jnp.float32)
        ivec = jnp.zeros((16,), jnp.float32)
        for t in range(NTILES):
            mvec = jnp.maximum(mvec, comb[pl.ds(t * 32, 16)])
            ivec = jnp.maximum(ivec, comb[pl.ds(t * 32 + 16, 16)])
        mg = jnp.max(mvec)
        ig = jnp.min(jnp.where(mvec == mg, ivec, BIG))

        valid = mg > -BIG
        vf = valid.astype(jnp.float32)
        igv = jnp.full((16,), ig, jnp.float32)
        iiv = igv.astype(jnp.int32)
        nb0 = plsc.load_gather(cx1, [iiv])
        nb1 = plsc.load_gather(cy1, [iiv])
        nb2 = plsc.load_gather(cx2, [iiv])
        nb3 = plsc.load_gather(cy2, [iiv])
        sc = mg * vf

        row = jnp.where(li == 0, nb0 * vf,
              jnp.where(li == 1, nb1 * vf,
              jnp.where(li == 2, nb2 * vf,
              jnp.where(li == 3, nb3 * vf,
              jnp.where(li == 5, sc, 0.0)))))
        outv[pl.ds(s * 16, 16)] = row

        return (igv, nb0, nb1, nb2, nb3)

    zv = jnp.zeros((16,), jnp.float32)
    init = (jnp.full((16,), -1.0, jnp.float32), zv, zv, zv, zv)
    lax.fori_loop(0, MAX_OUT, step, init)

    @pl.when(tid == 0)
    def _():
        pltpu.sync_copy(outv, out_hbm)


@jax.jit
def kernel(boxes, scores):
    bt = jnp.zeros((4, NPAD), jnp.float32).at[:, :N].set(boxes.T)
    sp = jnp.full((NPAD,), NEG_INF, jnp.float32).at[:N].set(scores)

    mesh = plsc.VectorSubcoreMesh(
        core_axis_name="c", subcore_axis_name="s",
        num_cores=1, num_subcores=NTILES)
    out = pl.kernel(
        _nms_body,
        out_type=jax.ShapeDtypeStruct((MAX_OUT * 16,), jnp.float32),
        mesh=mesh,
        compiler_params=pltpu.CompilerParams(needs_layout_passes=False),
        scratch_types=[
            pltpu.VMEM((NPAD,), jnp.float32),
            pltpu.VMEM((NPAD,), jnp.float32),
            pltpu.VMEM((NPAD,), jnp.float32),
            pltpu.VMEM((NPAD,), jnp.float32),
            pltpu.VMEM((PER_TILE,), jnp.float32),
            pltpu.VMEM((PER_TILE,), jnp.float32),
            pltpu.VMEM((32,), jnp.float32),
            pltpu.VMEM((512,), jnp.float32),
            pltpu.VMEM((MAX_OUT * 16,), jnp.float32),
            pltpu.VMEM_SHARED((1024,), jnp.float32),
        ],
    )(bt, sp)

    rows = out.reshape(MAX_OUT, 16)
    return rows[:, :6][None, :, :]
